# baseline (speedup 1.0000x reference)
import functools

import jax
import jax.numpy as jnp
from jax import lax
from jax.experimental import pallas as pl
from jax.experimental.pallas import tpu as pltpu
from jax.experimental.pallas import tpu_sc as plsc

NUM_EMB = 1_000_000
DIM = 32
BATCH = 16384
N_FIELDS = 26
TOTAL = BATCH * N_FIELDS

NC = 2
NS = 16
NW = NC * NS
IDX_PER_TILE = TOTAL // NW
CHUNK = 128
G = 1664
NCHUNK = G // CHUNK
NGROUP = IDX_PER_TILE // G

_mesh = plsc.VectorSubcoreMesh(core_axis_name="c", subcore_axis_name="s")


@functools.partial(
    pl.kernel,
    mesh=_mesh,
    out_type=jax.ShapeDtypeStruct((TOTAL, DIM), jnp.float32),
    compiler_params=pltpu.CompilerParams(use_tc_tiling_on_sc=False),
    scratch_types=[
        pltpu.VMEM((IDX_PER_TILE,), jnp.int32),
        pltpu.VMEM((2, G, DIM), jnp.float32),
        pltpu.SemaphoreType.DMA,
        pltpu.SemaphoreType.DMA,
        pltpu.SemaphoreType.DMA,
        pltpu.SemaphoreType.DMA,
    ],
)
def _gather_sc(idx_hbm, table_hbm, out_hbm, idx_v, buf, gsem0, gsem1,
               ssem0, ssem1):
    wid = lax.axis_index("s") * NC + lax.axis_index("c")
    base = wid * IDX_PER_TILE
    gsems = (gsem0, gsem1)
    ssems = (ssem0, ssem1)

    pltpu.sync_copy(idx_hbm.at[pl.ds(base, IDX_PER_TILE)], idx_v)

    def start_gathers(g, h):
        def body(c, carry):
            pltpu.async_copy(
                table_hbm.at[idx_v.at[pl.ds(g * G + c * CHUNK, CHUNK)]],
                buf.at[h].at[pl.ds(c * CHUNK, CHUNK)],
                gsems[h],
            )
            return carry
        lax.fori_loop(0, NCHUNK, body, 0)

    def wait_gathers(h):
        pltpu.make_async_copy(out_hbm.at[pl.ds(0, G)], buf.at[h],
                              gsems[h]).wait()

    def start_scatter(g, h):
        pltpu.async_copy(buf.at[h], out_hbm.at[pl.ds(base + g * G, G)],
                         ssems[h])

    def wait_scatter(h):
        pltpu.make_async_copy(buf.at[h], out_hbm.at[pl.ds(0, G)],
                              ssems[h]).wait()

    start_gathers(0, 0)
    for g in range(NGROUP):
        h = g % 2
        if g + 1 < NGROUP:
            if g >= 1:
                wait_scatter(1 - h)
            start_gathers(g + 1, 1 - h)
        wait_gathers(h)
        start_scatter(g, h)
    wait_scatter(0)
    wait_scatter(1)


def kernel(indices, table):
    idx_flat = lax.optimization_barrier(
        indices.astype(jnp.int32).reshape(TOTAL))
    tab2 = lax.optimization_barrier(table.reshape(NUM_EMB * DIM // 128, 128))
    out = _gather_sc(idx_flat, tab2.reshape(NUM_EMB, DIM))
    out2 = lax.optimization_barrier(out.reshape(TOTAL * DIM // 128, 128))
    return out2.reshape(BATCH, N_FIELDS, DIM)


# speedup vs baseline: 1.0006x; 1.0006x over previous
import functools

import jax
import jax.numpy as jnp
from jax import lax
from jax.experimental import pallas as pl
from jax.experimental.pallas import tpu as pltpu
from jax.experimental.pallas import tpu_sc as plsc

NUM_EMB = 1_000_000
DIM = 32
BATCH = 16384
N_FIELDS = 26
TOTAL = BATCH * N_FIELDS

NC = 2
NS = 16
NW = NC * NS
IDX_PER_TILE = TOTAL // NW
CHUNK = 128
G = 1664
NCHUNK = G // CHUNK
NGROUP = IDX_PER_TILE // G

_mesh = plsc.VectorSubcoreMesh(core_axis_name="c", subcore_axis_name="s")


@functools.partial(
    pl.kernel,
    mesh=_mesh,
    out_type=jax.ShapeDtypeStruct((TOTAL, DIM), jnp.float32),
    compiler_params=pltpu.CompilerParams(use_tc_tiling_on_sc=False),
    scratch_types=[
        pltpu.VMEM((IDX_PER_TILE,), jnp.int32),
        pltpu.VMEM((2, G, DIM), jnp.float32),
        pltpu.SemaphoreType.DMA,
        pltpu.SemaphoreType.DMA,
        pltpu.SemaphoreType.DMA,
        pltpu.SemaphoreType.DMA,
    ],
)
def _gather_sc(idx_hbm, table_hbm, out_hbm, idx_v, buf, gsem0, gsem1,
               ssem0, ssem1):
    wid = lax.axis_index("s") * NC + lax.axis_index("c")
    base = wid * IDX_PER_TILE
    gsems = (gsem0, gsem1)
    ssems = (ssem0, ssem1)

    pltpu.sync_copy(idx_hbm.at[pl.ds(base, IDX_PER_TILE)], idx_v)

    def start_gathers(g, h):
        def body(c, carry):
            pltpu.async_copy(
                table_hbm.at[idx_v.at[pl.ds(g * G + c * CHUNK, CHUNK)]],
                buf.at[h].at[pl.ds(c * CHUNK, CHUNK)],
                gsems[h],
            )
            return carry
        lax.fori_loop(0, NCHUNK, body, 0)

    def wait_gathers(h):
        pltpu.make_async_copy(out_hbm.at[pl.ds(0, G)], buf.at[h],
                              gsems[h]).wait()

    def start_scatter(g, h):
        pltpu.async_copy(buf.at[h], out_hbm.at[pl.ds(base + g * G, G)],
                         ssems[h])

    def wait_scatter(h):
        pltpu.make_async_copy(buf.at[h], out_hbm.at[pl.ds(0, G)],
                              ssems[h]).wait()

    start_gathers(0, 0)
    for g in range(NGROUP):
        h = g % 2
        if g + 1 < NGROUP:
            if g >= 1:
                wait_scatter(1 - h)
            start_gathers(g + 1, 1 - h)
        wait_gathers(h)
        start_scatter(g, h)
    wait_scatter(0)
    wait_scatter(1)


def kernel(indices, table):
    idx_flat = indices.astype(jnp.int32).reshape(TOTAL)
    out = _gather_sc(idx_flat, table)
    return out.reshape(BATCH, N_FIELDS, DIM)
